# baseline (speedup 1.0000x reference)
import functools

import jax
import jax.numpy as jnp
from jax import lax
from jax.experimental import pallas as pl
from jax.experimental.pallas import tpu as pltpu
from jax.experimental.pallas import tpu_sc as plsc

V = 1000000
D = 32
B = 16384
NC = 2
NS = 16
L = 16
NW = NC * NS
BPW = B // NW
BLOCKS = BPW // L
G = 4
NG = BPW // G
PAD = BPW + L

_mesh = plsc.VectorSubcoreMesh(core_axis_name="c", subcore_axis_name="s")


@functools.partial(
    pl.kernel,
    out_type=jax.ShapeDtypeStruct((B,), jnp.float32),
    mesh=_mesh,
    scratch_types=[
        pltpu.VMEM((2 * BPW,), jnp.int32),
        pltpu.VMEM((BPW,), jnp.int32),
        pltpu.VMEM((BPW,), jnp.int32),
        pltpu.VMEM((PAD,), jnp.int32),
        pltpu.VMEM((PAD,), jnp.int32),
        pltpu.VMEM((PAD,), jnp.int32),
        pltpu.VMEM((PAD,), jnp.int32),
        pltpu.VMEM((3 * G * D, 128), jnp.float32),
        pltpu.VMEM((3 * G * D, 128), jnp.float32),
        pltpu.VMEM((BPW,), jnp.float32),
        pltpu.VMEM((BPW,), jnp.float32),
        pltpu.VMEM((PAD,), jnp.float32),
        pltpu.SemaphoreType.DMA,
        pltpu.SemaphoreType.DMA,
        pltpu.SemaphoreType.DMA,
        pltpu.SemaphoreType.DMA,
    ],
    compiler_params=pltpu.CompilerParams(
        needs_layout_passes=False, use_tc_tiling_on_sc=True),
)
def _glove_sc(idx_flat_hbm, wT_c_hbm, wT_x_hbm,
              b_center_hbm, b_contex_hbm, out_hbm,
              idx2_v, idx_c_v, idx_x_v, coff_c_v, coff_x_v, lane_c_v, lane_x_v,
              cbuf_v, xbuf_v, bias_c_v, bias_x_v, out_v,
              sem0, sem1, sem2, semb):
    wid = lax.axis_index("s") * NC + lax.axis_index("c")
    base = wid * BPW

    pltpu.sync_copy(idx_flat_hbm.at[pl.ds(2 * base, 2 * BPW)], idx2_v)

    lanes16 = lax.iota(jnp.int32, L)

    def deint_body(blk, carry):
        p0 = blk * L
        even = 2 * (p0 + lanes16)
        ic = plsc.load_gather(idx2_v, [even])
        ix = plsc.load_gather(idx2_v, [even + 1])
        idx_c_v[pl.ds(p0, L)] = ic
        idx_x_v[pl.ds(p0, L)] = ix
        coff_c_v[pl.ds(p0, L)] = (ic >> 7) << 7
        coff_x_v[pl.ds(p0, L)] = (ix >> 7) << 7
        lane_c_v[pl.ds(p0, L)] = ic & 127
        lane_x_v[pl.ds(p0, L)] = ix & 127
        return carry

    lax.fori_loop(0, BLOCKS, deint_body, 0)

    g_bc = pltpu.async_copy(b_center_hbm.at[idx_c_v], bias_c_v, semb)
    g_bx = pltpu.async_copy(b_contex_hbm.at[idx_x_v], bias_x_v, semb)

    def issue(g, slot, sem):
        cvec = coff_c_v[pl.ds(G * g, L)]
        xvec = coff_x_v[pl.ds(G * g, L)]
        for u in range(G):
            r0 = (slot * G + u) * D
            co = pl.multiple_of(cvec[u], 128)
            xo = pl.multiple_of(xvec[u], 128)
            for t in range(D // 8):
                pltpu.async_copy(
                    wT_c_hbm.at[pl.ds(8 * t, 8), pl.ds(co, 128)],
                    cbuf_v.at[pl.ds(r0 + 8 * t, 8), :], sem)
                pltpu.async_copy(
                    wT_x_hbm.at[pl.ds(8 * t, 8), pl.ds(xo, 128)],
                    xbuf_v.at[pl.ds(r0 + 8 * t, 8), :], sem)

    def drain(slot, sem):
        for u in range(G):
            r0 = (slot * G + u) * D
            pltpu.make_async_copy(
                wT_c_hbm.at[:, pl.ds(0, 128)],
                cbuf_v.at[pl.ds(r0, D), :], sem).wait()
            pltpu.make_async_copy(
                wT_x_hbm.at[:, pl.ds(0, 128)],
                xbuf_v.at[pl.ds(r0, D), :], sem).wait()

    def extract(g, slot):
        lcvec = lane_c_v[pl.ds(G * g, L)]
        lxvec = lane_x_v[pl.ds(G * g, L)]
        accblk = jnp.zeros((L,), jnp.float32)
        for u in range(G):
            r0 = (slot * G + u) * D
            lc = lcvec[u]
            lx = lxvec[u]
            lc16 = (lc >> 4) << 4
            lx16 = (lx >> 4) << 4
            lcl = jnp.full((L,), lc & 15, jnp.int32)
            lxl = jnp.full((L,), lx & 15, jnp.int32)
            acc = jnp.zeros((L,), jnp.float32)
            for f in range(D):
                cv = cbuf_v[r0 + f, pl.ds(lc16, L)]
                xv = xbuf_v[r0 + f, pl.ds(lx16, L)]
                cb = cv.at[lcl].get(mode="promise_in_bounds")
                xb = xv.at[lxl].get(mode="promise_in_bounds")
                acc = acc + cb * xb
            accblk = jnp.where(lanes16 == u, acc, accblk)
        out_v[pl.ds(G * g, L)] = accblk

    sems = (sem0, sem1, sem2)
    issue(0, 0, sem0)
    issue(1, 1, sem1)

    def main_body(k, carry):
        for off in range(3):
            g = 3 * k + off
            slot = (off + 2) % 3
            issue(g + 2, slot, sems[slot])
            drain(off, sems[off])
            extract(g, off)
        return carry

    lax.fori_loop(0, (NG - 2) // 3, main_body, 0)
    drain(0, sem0)
    extract(NG - 2, 0)
    drain(1, sem1)
    extract(NG - 1, 1)

    g_bc.wait()
    g_bx.wait()

    def bias_body(blk, carry):
        p0 = blk * L
        out_v[pl.ds(p0, L)] = (out_v[pl.ds(p0, L)]
                               + bias_c_v[pl.ds(p0, L)]
                               + bias_x_v[pl.ds(p0, L)])
        return carry

    lax.fori_loop(0, BLOCKS, bias_body, 0)

    pltpu.sync_copy(out_v.at[pl.ds(0, BPW)], out_hbm.at[pl.ds(base, BPW)])


@jax.jit
def kernel(indices, w_center, w_contex, b_center, b_contex):
    idx_flat = indices.reshape(2 * B)
    return _glove_sc(idx_flat, w_center.T, w_contex.T, b_center, b_contex)
